# baseline (speedup 1.0000x reference)
import functools

import jax
import jax.numpy as jnp
from jax import lax
from jax.experimental import pallas as pl
from jax.experimental.pallas import tpu as pltpu
from jax.experimental.pallas import tpu_sc as plsc

_NC = 2
_NS = 16
_NW = _NC * _NS
_MSH = 25


def _tc_body(x_ref, w8_ref, b8_ref, ext_ref, wct_ref, cb_ref,
             nt_ref, et_ref):
    @pl.when(pl.program_id(0) == 0)
    def _():
        y = jax.lax.dot_general(
            w8_ref[...], x_ref[...],
            (((0,), (1,)), ((), ())),
            preferred_element_type=jnp.float32) + b8_ref[...]
        row = jax.lax.broadcasted_iota(jnp.int32, y.shape, 0)
        sig = 1.0 / (1.0 + jnp.exp(-y))
        nt_ref[...] = jnp.where((row == 2) | (row == 3), sig, y)

    et_ref[...] = (
        jnp.dot(wct_ref[...], ext_ref[...], preferred_element_type=jnp.float32)
        + cb_ref[...]
    )


def kernel(x, edge_x, edge_index, actual_amount, per_graph_size, We, be, Wa, ba, Wt, bt):
    N, D = x.shape
    E, DE = edge_x.shape

    Wra = Wa[:, :D].T
    Wca = Wa[:, D:2 * D].T
    Wea = Wa[:, 2 * D:].T
    Wc = We.T @ Wea
    ec = (be @ Wea + ba)[:, None]
    W8 = jnp.concatenate(
        [Wra, Wt.T, Wca, jnp.zeros((D, 2), jnp.float32)], axis=1)
    bias8 = jnp.concatenate(
        [jnp.zeros((2,), jnp.float32), bt, jnp.zeros((4,), jnp.float32)])
    bias8c = bias8[:, None]
    ext = edge_x.T
    wct = Wc.T

    pgs = jnp.asarray(per_graph_size, dtype=jnp.int32)
    magic = ((jnp.int32(1) << _MSH) + pgs - 1) // pgs
    magic16 = jnp.full((16,), 1, dtype=jnp.int32) * magic

    BEL = 64000
    nodetab, et = pl.pallas_call(
        _tc_body,
        grid=(E // BEL,),
        in_specs=[
            pl.BlockSpec((N, D), lambda i: (0, 0)),
            pl.BlockSpec((D, 8), lambda i: (0, 0)),
            pl.BlockSpec((8, 1), lambda i: (0, 0)),
            pl.BlockSpec((DE, BEL), lambda i: (0, i)),
            pl.BlockSpec((2, DE), lambda i: (0, 0)),
            pl.BlockSpec((2, 1), lambda i: (0, 0)),
        ],
        out_specs=[
            pl.BlockSpec((8, N), lambda i: (0, 0)),
            pl.BlockSpec((2, BEL), lambda i: (0, i)),
        ],
        out_shape=[
            jax.ShapeDtypeStruct((8, N), jnp.float32),
            jax.ShapeDtypeStruct((2, E), jnp.float32),
        ],
    )(x, W8, bias8c, ext, wct, ec)

    amt_t = actual_amount.T

    EPW = E // _NW
    C = 2000
    NCH = EPW // C
    U = 5
    CA1 = 2176
    CA2 = 10112
    assert EPW % C == 0 and C % (16 * U) == 0

    mesh = plsc.VectorSubcoreMesh(core_axis_name="c", subcore_axis_name="s")

    @functools.partial(
        pl.kernel,
        out_type=(
            jax.ShapeDtypeStruct((E,), jnp.float32),
            jax.ShapeDtypeStruct((E,), jnp.float32),
            jax.ShapeDtypeStruct((_NW * 32,), jnp.float32),
        ),
        mesh=mesh,
        compiler_params=pltpu.CompilerParams(needs_layout_passes=False),
        scratch_types=[
            pltpu.VMEM((8, N), jnp.float32),
            pltpu.VMEM((2, CA1), jnp.int32), pltpu.VMEM((2, CA1), jnp.int32),
            pltpu.VMEM((2, CA1), jnp.float32), pltpu.VMEM((2, CA1), jnp.float32),
            pltpu.VMEM((2, CA1), jnp.float32), pltpu.VMEM((2, CA1), jnp.float32),
            pltpu.VMEM((C,), jnp.float32), pltpu.VMEM((C,), jnp.float32),
            pltpu.VMEM((C,), jnp.float32), pltpu.VMEM((C,), jnp.float32),
            pltpu.VMEM((256,), jnp.float32),
            pltpu.VMEM((256,), jnp.float32),
            pltpu.VMEM((32,), jnp.float32),
            pltpu.VMEM((16,), jnp.int32),
            pltpu.SemaphoreType.DMA, pltpu.SemaphoreType.DMA,
            pltpu.SemaphoreType.DMA, pltpu.SemaphoreType.DMA,
        ],
    )
    def _pass1(nt_h, ei_h, et_h, amt_h, mg_h,
               p0_h, p1_h, part_h,
               tab_v, ei_a, ei_b, et_a, et_b, am_a, am_b,
               p0a, p0b, p1a, p1b,
               acc0_v, acc1_v, accw_v, mg_v,
               isem_a, isem_b, osem_a, osem_b):
        wid = lax.axis_index("s") * _NC + lax.axis_index("c")
        base_w = wid * EPW
        eis = [ei_a, ei_b]
        ets = [et_a, et_b]
        ams = [am_a, am_b]
        p0s = [p0a, p0b]
        p1s = [p1a, p1b]
        isems = [isem_a, isem_b]
        osems = [osem_a, osem_b]

        def chunk_base(k):
            base = base_w + k * C
            abase = jnp.minimum((base // 128) * 128, E - CA1)
            return base, abase

        def in_copies(k):
            s = k % 2
            base, abase = chunk_base(k)
            sl = pl.ds(abase, CA1)
            rs = pl.ds(0, 2)
            return [
                pltpu.make_async_copy(ei_h.at[rs, sl], eis[s], isems[s]),
                pltpu.make_async_copy(et_h.at[rs, sl], ets[s], isems[s]),
                pltpu.make_async_copy(amt_h.at[rs, sl], ams[s], isems[s]),
            ]

        def out_copies(k):
            s = k % 2
            base = base_w + k * C
            return [
                pltpu.make_async_copy(p0s[s], p0_h.at[pl.ds(base, C)], osems[s]),
                pltpu.make_async_copy(p1s[s], p1_h.at[pl.ds(base, C)], osems[s]),
            ]

        for c in in_copies(0):
            c.start()
        pltpu.sync_copy(nt_h, tab_v)
        pltpu.sync_copy(mg_h, mg_v)
        zf = jnp.zeros((16,), jnp.float32)
        for i in range(16):
            acc0_v[pl.ds(i * 16, 16)] = zf
            acc1_v[pl.ds(i * 16, 16)] = zf
        lane = lax.iota(jnp.int32, 16)
        mg = mg_v[:]

        for k in range(NCH):
            s = k % 2
            if k + 1 < NCH:
                for c in in_copies(k + 1):
                    c.start()
            for c in in_copies(k):
                c.wait()
            if k >= 2:
                for c in out_copies(k - 2):
                    c.wait()
            ei_v, et_v, am_v = eis[s], ets[s], ams[s]
            p0_v, p1_v = p0s[s], p1s[s]
            base, abase = chunk_base(k)
            off = base - abase

            @plsc.parallel_loop(0, C // 16, unroll=U)
            def body(j):
                if True:
                    o = j * 16
                    oo = off + o
                    r16 = ei_v[0, pl.ds(oo, 16)]
                    c16 = ei_v[1, pl.ds(oo, 16)]
                    zi = jnp.zeros((16,), jnp.int32)
                    a0 = plsc.load_gather(tab_v, [zi, r16])
                    a1 = plsc.load_gather(tab_v, [zi + 1, r16])
                    t0 = plsc.load_gather(tab_v, [zi + 2, r16])
                    t1 = plsc.load_gather(tab_v, [zi + 3, r16])
                    b0 = plsc.load_gather(tab_v, [zi + 4, c16])
                    b1 = plsc.load_gather(tab_v, [zi + 5, c16])
                    z0 = a0 + b0 + et_v[0, pl.ds(oo, 16)]
                    z1 = a1 + b1 + et_v[1, pl.ds(oo, 16)]
                    s0 = 1.0 / (1.0 + jnp.exp(-z0))
                    s1 = 1.0 / (1.0 + jnp.exp(-z1))
                    att0 = jnp.maximum(s0 - t0, 0.0)
                    att1 = jnp.maximum(s1 - t1, 0.0)
                    bi = jnp.right_shift(r16 * mg, _MSH)
                    li = lane * 16 + bi
                    plsc.addupdate_scatter(acc0_v, [li], att0)
                    plsc.addupdate_scatter(acc1_v, [li], att1)
                    p0_v[pl.ds(o, 16)] = att0 * am_v[0, pl.ds(oo, 16)]
                    p1_v[pl.ds(o, 16)] = att1 * am_v[1, pl.ds(oo, 16)]

            for c in out_copies(k):
                c.start()

        for c in out_copies(NCH - 2):
            c.wait()
        for c in out_copies(NCH - 1):
            c.wait()
        s0 = acc0_v[pl.ds(0, 16)]
        s1 = acc1_v[pl.ds(0, 16)]
        for i in range(1, 16):
            s0 = s0 + acc0_v[pl.ds(i * 16, 16)]
            s1 = s1 + acc1_v[pl.ds(i * 16, 16)]
        accw_v[pl.ds(0, 16)] = s0
        accw_v[pl.ds(16, 16)] = s1
        pltpu.sync_copy(accw_v, part_h.at[pl.ds(wid * 32, 32)])

    p0, p1, part = _pass1(nodetab, edge_index, et, amt_t, magic16)

    @functools.partial(
        pl.kernel,
        out_type=jax.ShapeDtypeStruct((E,), jnp.float32),
        mesh=mesh,
        compiler_params=pltpu.CompilerParams(needs_layout_passes=False),
        scratch_types=[
            pltpu.VMEM((_NW * 2 * 16,), jnp.float32),
            pltpu.VMEM((EPW,), jnp.float32),
            pltpu.VMEM((EPW,), jnp.float32),
            pltpu.VMEM((2, CA2), jnp.int32),
            pltpu.VMEM((EPW,), jnp.float32),
            pltpu.VMEM((16,), jnp.float32),
            pltpu.VMEM((16,), jnp.float32),
            pltpu.VMEM((16,), jnp.int32),
            pltpu.SemaphoreType.DMA,
        ],
    )
    def _pass2(p0_h, p1_h, ei_h, part_h, mg_h, out_h,
               part_v, p0_v, p1_v, row_v, out_v, rbs0_v, rbs1_v, mg_v, isem):
        wid = lax.axis_index("s") * _NC + lax.axis_index("c")
        base_w = wid * EPW
        abase = jnp.minimum((base_w // 128) * 128, E - CA2)
        off = base_w - abase
        ins = [
            pltpu.make_async_copy(p0_h.at[pl.ds(base_w, EPW)], p0_v, isem),
            pltpu.make_async_copy(p1_h.at[pl.ds(base_w, EPW)], p1_v, isem),
            pltpu.make_async_copy(
                ei_h.at[pl.ds(0, 2), pl.ds(abase, CA2)], row_v, isem),
        ]
        for c in ins:
            c.start()
        pltpu.sync_copy(mg_h, mg_v)
        pltpu.sync_copy(part_h, part_v)
        s0 = jnp.zeros((16,), jnp.float32)
        s1 = jnp.zeros((16,), jnp.float32)
        for w in range(_NW):
            s0 = s0 + part_v[pl.ds(w * 32, 16)]
            s1 = s1 + part_v[pl.ds(w * 32 + 16, 16)]
        rbs0_v[:] = 1.0 / (s0 + 1e-5)
        rbs1_v[:] = 1.0 / (s1 + 1e-5)
        mg = mg_v[:]
        for c in ins:
            c.wait()

        @plsc.parallel_loop(0, EPW // 16, unroll=U)
        def body(j):
            o = j * 16
            bi = jnp.right_shift(row_v[0, pl.ds(off + o, 16)] * mg, _MSH)
            r0 = plsc.load_gather(rbs0_v, [bi])
            r1 = plsc.load_gather(rbs1_v, [bi])
            out_v[pl.ds(o, 16)] = (
                p0_v[pl.ds(o, 16)] * r0 + p1_v[pl.ds(o, 16)] * r1)
        pltpu.sync_copy(out_v, out_h.at[pl.ds(base_w, EPW)])

    return _pass2(p0, p1, edge_index, part, magic16)


# speedup vs baseline: 1.0216x; 1.0216x over previous
import functools

import jax
import jax.numpy as jnp
from jax import lax
from jax.experimental import pallas as pl
from jax.experimental.pallas import tpu as pltpu
from jax.experimental.pallas import tpu_sc as plsc

_NC = 2
_NS = 16
_NW = _NC * _NS
_MSH = 25


def _tc_body(x_ref, wa_ref, wt_ref, bt_ref, we_ref, be_ref, ba_ref, ext_ref,
             nt_ref, et_ref):
    D = x_ref.shape[1]
    HE = we_ref.shape[0]
    dn = (((1,), (1,)), ((), ()))
    wae = wa_ref[:, 2 * D:2 * D + HE]
    ec = jnp.sum(wae * be_ref[...], axis=1, keepdims=True) + ba_ref[...]

    @pl.when(pl.program_id(0) == 0)
    def _():
        xv = x_ref[...]
        ya = jax.lax.dot_general(wa_ref[:, 0:D], xv, dn,
                                 preferred_element_type=jnp.float32) + ec
        yb = jax.lax.dot_general(wa_ref[:, D:2 * D], xv, dn,
                                 preferred_element_type=jnp.float32)
        yt = jax.lax.dot_general(wt_ref[...], xv, dn,
                                 preferred_element_type=jnp.float32) + bt_ref[...]
        yt = 1.0 / (1.0 + jnp.exp(-yt))
        nt_ref[...] = jnp.concatenate([ya, yt, yb, jnp.zeros_like(ya)], axis=0)

    wct = jnp.dot(wae, we_ref[...], preferred_element_type=jnp.float32)
    et_ref[...] = (
        jnp.dot(wct, ext_ref[...], preferred_element_type=jnp.float32) + ec
    )


def kernel(x, edge_x, edge_index, actual_amount, per_graph_size, We, be, Wa, ba, Wt, bt):
    N, D = x.shape
    E, DE = edge_x.shape

    ext = edge_x.T
    be1 = be[None, :]
    ba1 = ba[:, None]
    bt1 = bt[:, None]
    HE = We.shape[0]

    pgs = jnp.asarray(per_graph_size, dtype=jnp.int32)
    magic = ((jnp.int32(1) << _MSH) + pgs - 1) // pgs
    magic16 = jnp.full((16,), 1, dtype=jnp.int32) * magic

    BEL = 64000
    nodetab, et = pl.pallas_call(
        _tc_body,
        grid=(E // BEL,),
        in_specs=[
            pl.BlockSpec((N, D), lambda i: (0, 0)),
            pl.BlockSpec((2, 2 * D + HE), lambda i: (0, 0)),
            pl.BlockSpec((2, D), lambda i: (0, 0)),
            pl.BlockSpec((2, 1), lambda i: (0, 0)),
            pl.BlockSpec((HE, DE), lambda i: (0, 0)),
            pl.BlockSpec((1, HE), lambda i: (0, 0)),
            pl.BlockSpec((2, 1), lambda i: (0, 0)),
            pl.BlockSpec((DE, BEL), lambda i: (0, i)),
        ],
        out_specs=[
            pl.BlockSpec((8, N), lambda i: (0, 0)),
            pl.BlockSpec((2, BEL), lambda i: (0, i)),
        ],
        out_shape=[
            jax.ShapeDtypeStruct((8, N), jnp.float32),
            jax.ShapeDtypeStruct((2, E), jnp.float32),
        ],
    )(x, Wa, Wt, bt1, We, be1, ba1, ext)

    amt_t = actual_amount.T

    EPW = E // _NW
    C = 2000
    NCH = EPW // C
    U = 5
    CA1 = 2176
    CA2 = 10112
    assert EPW % C == 0 and C % (16 * U) == 0

    mesh = plsc.VectorSubcoreMesh(core_axis_name="c", subcore_axis_name="s")

    @functools.partial(
        pl.kernel,
        out_type=(
            jax.ShapeDtypeStruct((E,), jnp.float32),
            jax.ShapeDtypeStruct((E,), jnp.float32),
            jax.ShapeDtypeStruct((_NW * 32,), jnp.float32),
        ),
        mesh=mesh,
        compiler_params=pltpu.CompilerParams(needs_layout_passes=False),
        scratch_types=[
            pltpu.VMEM((8, N), jnp.float32),
            pltpu.VMEM((2, CA1), jnp.int32), pltpu.VMEM((2, CA1), jnp.int32),
            pltpu.VMEM((2, CA1), jnp.float32), pltpu.VMEM((2, CA1), jnp.float32),
            pltpu.VMEM((2, CA1), jnp.float32), pltpu.VMEM((2, CA1), jnp.float32),
            pltpu.VMEM((C,), jnp.float32), pltpu.VMEM((C,), jnp.float32),
            pltpu.VMEM((C,), jnp.float32), pltpu.VMEM((C,), jnp.float32),
            pltpu.VMEM((256,), jnp.float32),
            pltpu.VMEM((256,), jnp.float32),
            pltpu.VMEM((32,), jnp.float32),
            pltpu.VMEM((16,), jnp.int32),
            pltpu.SemaphoreType.DMA, pltpu.SemaphoreType.DMA,
            pltpu.SemaphoreType.DMA, pltpu.SemaphoreType.DMA,
        ],
    )
    def _pass1(nt_h, ei_h, et_h, amt_h, mg_h,
               p0_h, p1_h, part_h,
               tab_v, ei_a, ei_b, et_a, et_b, am_a, am_b,
               p0a, p0b, p1a, p1b,
               acc0_v, acc1_v, accw_v, mg_v,
               isem_a, isem_b, osem_a, osem_b):
        wid = lax.axis_index("s") * _NC + lax.axis_index("c")
        base_w = wid * EPW
        eis = [ei_a, ei_b]
        ets = [et_a, et_b]
        ams = [am_a, am_b]
        p0s = [p0a, p0b]
        p1s = [p1a, p1b]
        isems = [isem_a, isem_b]
        osems = [osem_a, osem_b]

        def chunk_base(k):
            base = base_w + k * C
            abase = jnp.minimum((base // 128) * 128, E - CA1)
            return base, abase

        def in_copies(k):
            s = k % 2
            base, abase = chunk_base(k)
            sl = pl.ds(abase, CA1)
            rs = pl.ds(0, 2)
            return [
                pltpu.make_async_copy(ei_h.at[rs, sl], eis[s], isems[s]),
                pltpu.make_async_copy(et_h.at[rs, sl], ets[s], isems[s]),
                pltpu.make_async_copy(amt_h.at[rs, sl], ams[s], isems[s]),
            ]

        def out_copies(k):
            s = k % 2
            base = base_w + k * C
            return [
                pltpu.make_async_copy(p0s[s], p0_h.at[pl.ds(base, C)], osems[s]),
                pltpu.make_async_copy(p1s[s], p1_h.at[pl.ds(base, C)], osems[s]),
            ]

        for c in in_copies(0):
            c.start()
        pltpu.sync_copy(nt_h, tab_v)
        pltpu.sync_copy(mg_h, mg_v)
        zf = jnp.zeros((16,), jnp.float32)
        for i in range(16):
            acc0_v[pl.ds(i * 16, 16)] = zf
            acc1_v[pl.ds(i * 16, 16)] = zf
        lane = lax.iota(jnp.int32, 16)
        mg = mg_v[:]

        for k in range(NCH):
            s = k % 2
            if k + 1 < NCH:
                for c in in_copies(k + 1):
                    c.start()
            for c in in_copies(k):
                c.wait()
            if k >= 2:
                for c in out_copies(k - 2):
                    c.wait()
            ei_v, et_v, am_v = eis[s], ets[s], ams[s]
            p0_v, p1_v = p0s[s], p1s[s]
            base, abase = chunk_base(k)
            off = base - abase

            @plsc.parallel_loop(0, C // 16, unroll=U)
            def body(j):
                if True:
                    o = j * 16
                    oo = off + o
                    r16 = ei_v[0, pl.ds(oo, 16)]
                    c16 = ei_v[1, pl.ds(oo, 16)]
                    zi = jnp.zeros((16,), jnp.int32)
                    a0 = plsc.load_gather(tab_v, [zi, r16])
                    a1 = plsc.load_gather(tab_v, [zi + 1, r16])
                    t0 = plsc.load_gather(tab_v, [zi + 2, r16])
                    t1 = plsc.load_gather(tab_v, [zi + 3, r16])
                    b0 = plsc.load_gather(tab_v, [zi + 4, c16])
                    b1 = plsc.load_gather(tab_v, [zi + 5, c16])
                    z0 = a0 + b0 + et_v[0, pl.ds(oo, 16)]
                    z1 = a1 + b1 + et_v[1, pl.ds(oo, 16)]
                    s0 = 1.0 / (1.0 + jnp.exp(-z0))
                    s1 = 1.0 / (1.0 + jnp.exp(-z1))
                    att0 = jnp.maximum(s0 - t0, 0.0)
                    att1 = jnp.maximum(s1 - t1, 0.0)
                    bi = jnp.right_shift(r16 * mg, _MSH)
                    li = lane * 16 + bi
                    plsc.addupdate_scatter(acc0_v, [li], att0)
                    plsc.addupdate_scatter(acc1_v, [li], att1)
                    p0_v[pl.ds(o, 16)] = att0 * am_v[0, pl.ds(oo, 16)]
                    p1_v[pl.ds(o, 16)] = att1 * am_v[1, pl.ds(oo, 16)]

            for c in out_copies(k):
                c.start()

        for c in out_copies(NCH - 2):
            c.wait()
        for c in out_copies(NCH - 1):
            c.wait()
        s0 = acc0_v[pl.ds(0, 16)]
        s1 = acc1_v[pl.ds(0, 16)]
        for i in range(1, 16):
            s0 = s0 + acc0_v[pl.ds(i * 16, 16)]
            s1 = s1 + acc1_v[pl.ds(i * 16, 16)]
        accw_v[pl.ds(0, 16)] = s0
        accw_v[pl.ds(16, 16)] = s1
        pltpu.sync_copy(accw_v, part_h.at[pl.ds(wid * 32, 32)])

    p0, p1, part = _pass1(nodetab, edge_index, et, amt_t, magic16)

    @functools.partial(
        pl.kernel,
        out_type=jax.ShapeDtypeStruct((E,), jnp.float32),
        mesh=mesh,
        compiler_params=pltpu.CompilerParams(needs_layout_passes=False),
        scratch_types=[
            pltpu.VMEM((_NW * 2 * 16,), jnp.float32),
            pltpu.VMEM((EPW,), jnp.float32),
            pltpu.VMEM((EPW,), jnp.float32),
            pltpu.VMEM((2, CA2), jnp.int32),
            pltpu.VMEM((EPW,), jnp.float32),
            pltpu.VMEM((16,), jnp.float32),
            pltpu.VMEM((16,), jnp.float32),
            pltpu.VMEM((16,), jnp.int32),
            pltpu.SemaphoreType.DMA,
        ],
    )
    def _pass2(p0_h, p1_h, ei_h, part_h, mg_h, out_h,
               part_v, p0_v, p1_v, row_v, out_v, rbs0_v, rbs1_v, mg_v, isem):
        wid = lax.axis_index("s") * _NC + lax.axis_index("c")
        base_w = wid * EPW
        abase = jnp.minimum((base_w // 128) * 128, E - CA2)
        off = base_w - abase
        ins = [
            pltpu.make_async_copy(p0_h.at[pl.ds(base_w, EPW)], p0_v, isem),
            pltpu.make_async_copy(p1_h.at[pl.ds(base_w, EPW)], p1_v, isem),
            pltpu.make_async_copy(
                ei_h.at[pl.ds(0, 2), pl.ds(abase, CA2)], row_v, isem),
        ]
        for c in ins:
            c.start()
        pltpu.sync_copy(mg_h, mg_v)
        pltpu.sync_copy(part_h, part_v)
        s0 = jnp.zeros((16,), jnp.float32)
        s1 = jnp.zeros((16,), jnp.float32)
        for w in range(_NW):
            s0 = s0 + part_v[pl.ds(w * 32, 16)]
            s1 = s1 + part_v[pl.ds(w * 32 + 16, 16)]
        rbs0_v[:] = 1.0 / (s0 + 1e-5)
        rbs1_v[:] = 1.0 / (s1 + 1e-5)
        mg = mg_v[:]
        for c in ins:
            c.wait()

        @plsc.parallel_loop(0, EPW // 16, unroll=U)
        def body(j):
            o = j * 16
            bi = jnp.right_shift(row_v[0, pl.ds(off + o, 16)] * mg, _MSH)
            r0 = plsc.load_gather(rbs0_v, [bi])
            r1 = plsc.load_gather(rbs1_v, [bi])
            out_v[pl.ds(o, 16)] = (
                p0_v[pl.ds(o, 16)] * r0 + p1_v[pl.ds(o, 16)] * r1)
        pltpu.sync_copy(out_v, out_h.at[pl.ds(base_w, EPW)])

    return _pass2(p0, p1, edge_index, part, magic16)
